# Initial kernel scaffold; baseline (speedup 1.0000x reference)
import jax
import jax.numpy as jnp
from jax.experimental import pallas as pl


def kernel(segment_ids, sam_encoder_embeddings):
    raise NotImplementedError("write your pallas kernel here")


# speedup vs baseline: 4.1067x; 4.1067x over previous
import jax
import jax.numpy as jnp
from jax.experimental import pallas as pl

NSEG = 64
MINPIX = 16.0


def _body(seg_ref, fm_ref, out_ref, cnt_ref):
    seg = seg_ref[0, 0, :]
    onehot = (seg[:, None] == jax.lax.broadcasted_iota(jnp.int32, (1, NSEG), 1)
              ).astype(jnp.float32)
    fm = fm_ref[0]
    sums = jnp.dot(fm, onehot, preferred_element_type=jnp.float32)
    counts = jnp.sum(onehot, axis=0)
    scale = jnp.where(counts >= MINPIX, 1.0 / jnp.maximum(counts, 1.0), 0.0)
    out_ref[0] = jnp.transpose(sums * scale[None, :])
    cnt_ref[0, 0] = counts


def kernel(segment_ids, sam_encoder_embeddings):
    fm = jnp.squeeze(sam_encoder_embeddings, axis=1)
    B, C, h, w = fm.shape
    P = h * w
    fm = fm.reshape(B, C, P)
    seg = segment_ids.reshape(B, 1, P)

    out, cnt = pl.pallas_call(
        _body,
        grid=(B,),
        in_specs=[
            pl.BlockSpec((1, 1, P), lambda b: (b, 0, 0)),
            pl.BlockSpec((1, C, P), lambda b: (b, 0, 0)),
        ],
        out_specs=[
            pl.BlockSpec((1, NSEG, C), lambda b: (b, 0, 0)),
            pl.BlockSpec((1, 1, NSEG), lambda b: (b, 0, 0)),
        ],
        out_shape=[
            jax.ShapeDtypeStruct((B, NSEG, C), jnp.float32),
            jax.ShapeDtypeStruct((B, 1, NSEG), jnp.float32),
        ],
    )(seg, fm)

    valid = cnt.reshape(B, NSEG) >= MINPIX
    return out, valid
